# Initial kernel scaffold; baseline (speedup 1.0000x reference)
import jax
import jax.numpy as jnp
from jax.experimental import pallas as pl


def kernel(x, edge_index, batch, W1, b1, g1w, g1b, g1m, W2, b2, g2w, g2b, g2m, W3, b3, g3w, g3b, g3m, lW, lb):
    raise NotImplementedError("write your pallas kernel here")


# speedup vs baseline: 21.4806x; 21.4806x over previous
import dataclasses
import functools

import jax
import jax.numpy as jnp
from jax import lax
from jax.experimental import pallas as pl
from jax.experimental.pallas import tpu as pltpu
from jax.experimental.pallas import tpu_sc as plsc

N = 100000
E = 3200000
G = 512
H = 32
HH = 16
EPS = 1e-5

NSUB = 16
EROWS = 25088
EPAD = EROWS * 128 - E
NPAD = 100096
DUMP = N
RT = 6272
RT15 = 5920
MACRO = 4

@functools.cache
def _mesh():
  return plsc.VectorSubcoreMesh(core_axis_name="c", subcore_axis_name="s")


def _sc_params():
  cp = pltpu.CompilerParams()
  cp = dataclasses.replace(cp, needs_layout_passes=False,
                           use_tc_tiling_on_sc=False)
  return cp


def _zero_vmem(ref, rows, cols):
  @pl.loop(0, rows)
  def _(i):
    @pl.loop(0, cols, step=16)
    def _(j):
      ref[i, pl.ds(j, 16)] = jnp.zeros((16,), jnp.float32)


def _rsqrt16(x):
  i = plsc.bitcast(x, jnp.int32)
  i = jnp.int32(0x5F3759DF) - lax.shift_right_logical(i, 1)
  r = plsc.bitcast(i, jnp.float32)
  for _ in range(3):
    r = r * (1.5 - 0.5 * x * r * r)
  return r


def _deg_body(dstp, degp, acc, idxb, ones, zb, sems):
  c = lax.axis_index("c")
  s = lax.axis_index("s")
  @pl.loop(0, RT, step=16)
  def _(j):
    zb[pl.ds(j, 16)] = jnp.zeros((16,), jnp.float32)

  @pl.when(s < NSUB - 1)
  def _():
    pltpu.sync_copy(zb, acc.at[pl.ds(s * RT, RT)])

  @pl.when(s == NSUB - 1)
  def _():
    pltpu.sync_copy(zb.at[pl.ds(0, NPAD - (NSUB - 1) * RT)],
                    acc.at[pl.ds((NSUB - 1) * RT, NPAD - (NSUB - 1) * RT)])
  plsc.subcore_barrier()
  wid = c * NSUB + s
  base = wid * (EROWS // 32)
  @pl.loop(0, (EROWS // 32) // MACRO)
  def _(m):
    pltpu.sync_copy(dstp.at[pl.ds(base + m * MACRO, MACRO)], idxb)
    descs = []
    for j in range(MACRO):
      descs.append(
          pltpu.async_copy(ones, acc.at[idxb.at[j]], sems, add=True))
    for d in descs:
      d.wait()
  plsc.subcore_barrier()

  @pl.when(s < NSUB - 1)
  def _():
    pltpu.sync_copy(acc.at[pl.ds(s * RT, RT)],
                    degp.at[c, 0, pl.ds(s * RT, RT)])

  @pl.when(s == NSUB - 1)
  def _():
    pltpu.sync_copy(acc.at[pl.ds((NSUB - 1) * RT, NPAD - (NSUB - 1) * RT)],
                    degp.at[c, 0, pl.ds((NSUB - 1) * RT,
                                        NPAD - (NSUB - 1) * RT)])


def _deg_kernel(dstp):
  ker = pl.kernel(
      _deg_body,
      out_type=jax.ShapeDtypeStruct((2, 1, NPAD), jnp.float32),
      mesh=_mesh(),
      compiler_params=_sc_params(),
      scratch_types=[
          pltpu.VMEM_SHARED((NPAD,), jnp.float32),
          pltpu.VMEM((MACRO, 128), jnp.int32),
          pltpu.VMEM((128,), jnp.float32),
          pltpu.VMEM((RT,), jnp.float32),
          pltpu.SemaphoreType.DMA,
      ],
  )
  return ker(dstp)


def _b1_body(x_ref, degp_ref, batch_ref, w_ref, y_ref, dinv_ref, invc_ref,
             cnt_ref):
  i = pl.program_id(0)

  @pl.when(i == 0)
  def _():
    cnt_ref[...] = jnp.zeros_like(cnt_ref)

  deg = degp_ref[0, 0, :] + degp_ref[0, 1, :] + 1.0
  dinv = lax.rsqrt(deg)
  dinv_ref[0, 0, :] = dinv
  xw = jnp.dot(x_ref[...], w_ref[...], preferred_element_type=jnp.float32)
  y = xw * dinv[:, None]
  y_ref[0, :, :] = y[:, :HH]
  y_ref[1, :, :] = y[:, HH:]
  mask = (batch_ref[0, 0, :][:, None] ==
          lax.broadcasted_iota(jnp.int32, (1, G), 1)).astype(jnp.float32)
  cnt_ref[...] += jnp.sum(mask, axis=0)

  @pl.when(i == pl.num_programs(0) - 1)
  def _():
    invc_ref[...] = 1.0 / cnt_ref[...]


def _b1(x, degp, batch, w1):
  R = 5000
  grid = N // R
  degp_t = degp.reshape(2, grid, R).transpose(1, 0, 2)
  batch2 = batch.reshape(grid, 1, R)
  y, dinv2, invc = pl.pallas_call(
      _b1_body,
      grid=(grid,),
      in_specs=[
          pl.BlockSpec((R, 3), lambda i: (i, 0)),
          pl.BlockSpec((1, 2, R), lambda i: (i, 0, 0)),
          pl.BlockSpec((1, 1, R), lambda i: (i, 0, 0)),
          pl.BlockSpec((3, H), lambda i: (0, 0)),
      ],
      out_specs=[
          pl.BlockSpec((2, R, HH), lambda i: (0, i, 0)),
          pl.BlockSpec((1, 1, R), lambda i: (i, 0, 0)),
          pl.BlockSpec((G,), lambda i: (0,)),
      ],
      out_shape=[
          jax.ShapeDtypeStruct((2, NPAD, HH), jnp.float32),
          jax.ShapeDtypeStruct((grid, 1, R), jnp.float32),
          jax.ShapeDtypeStruct((G,), jnp.float32),
      ],
      scratch_shapes=[pltpu.VMEM((G,), jnp.float32)],
  )(x, degp_t, batch2, w1)
  return y, dinv2.reshape(N), invc


def _b_body(x_ref, dinv_ref, w_ref, y_ref):
  xc = jnp.concatenate([x_ref[0, :, :], x_ref[1, :, :]], axis=1)
  xw = jnp.dot(xc, w_ref[...], preferred_element_type=jnp.float32)
  y = xw * dinv_ref[0, 0, :][:, None]
  y_ref[0, :, :] = y[:, :HH]
  y_ref[1, :, :] = y[:, HH:]


def _b(x2, dinv, w):
  R = 5000
  grid = N // R
  return pl.pallas_call(
      _b_body,
      grid=(grid,),
      in_specs=[
          pl.BlockSpec((2, R, HH), lambda i: (0, i, 0)),
          pl.BlockSpec((1, 1, R), lambda i: (i, 0, 0)),
          pl.BlockSpec((H, H), lambda i: (0, 0)),
      ],
      out_specs=pl.BlockSpec((2, R, HH), lambda i: (0, i, 0)),
      out_shape=jax.ShapeDtypeStruct((2, NPAD, HH), jnp.float32),
  )(x2, dinv.reshape(grid, 1, R), w)


def _f_body(poolp_ref, invc_ref, lw_ref, lb_ref, o_ref):
  pooled = jnp.concatenate([poolp_ref[0, :, :], poolp_ref[1, :, :]], axis=1)
  pooled = pooled * invc_ref[...][:, None]
  o_ref[...] = (jnp.dot(pooled, lw_ref[...],
                        preferred_element_type=jnp.float32) +
                lb_ref[...][None, :])


def _f(poolp, invc, lw, lb):
  return pl.pallas_call(
      _f_body,
      out_shape=jax.ShapeDtypeStruct((G, 3), jnp.float32),
  )(poolp, invc, lw, lb)


def _layer_body(has_res, do_pool, *refs):
  if has_res:
    (y, srcp, dstp, dinv, invc, batch, gp, xp, xo, *rest) = refs
  else:
    (y, srcp, dstp, dinv, invc, batch, gp, xo, *rest) = refs
    xp = None
  if do_pool:
    poolo = rest[0]
    rest = rest[1:]
  else:
    poolo = None
  (acc, mom, prm, psh, sidx, didx, rows3, vrow, hh2, dbc, bic128, bic32,
   pbuf, xpb, xb, mrows, icb, prmb, gpb, gsem, ssem) = rest

  c = lax.axis_index("c")
  s = lax.axis_index("s")

  @pl.when(s < NSUB - 1)
  def _():
    pltpu.sync_copy(y.at[c].at[pl.ds(s * RT, RT)], acc.at[pl.ds(s * RT, RT)])

  @pl.when(s == NSUB - 1)
  def _():
    pltpu.sync_copy(y.at[c].at[pl.ds((NSUB - 1) * RT, NPAD - (NSUB - 1) * RT)],
                    acc.at[pl.ds((NSUB - 1) * RT, NPAD - (NSUB - 1) * RT)])

  _zero_vmem(hh2, 32, 32)
  pltpu.sync_copy(hh2.at[pl.ds(0, 32)], mom.at[pl.ds(s * 32, 32)])
  if do_pool:
    _zero_vmem(xb, 32, 16)
    pltpu.sync_copy(xb.at[pl.ds(0, 32)], psh.at[pl.ds(s * 32, 32)])
  pltpu.sync_copy(gp.at[c], gpb)
  plsc.subcore_barrier()

  ebase = s * (EROWS // NSUB)
  @pl.loop(0, (EROWS // NSUB) // MACRO)
  def _(m):
    pltpu.sync_copy(srcp.at[pl.ds(ebase + m * MACRO, MACRO)], sidx)
    pltpu.sync_copy(dstp.at[pl.ds(ebase + m * MACRO, MACRO)], didx)
    gd = [pltpu.async_copy(y.at[c].at[sidx.at[j]], rows3.at[j], gsem)
          for j in range(MACRO)]
    for d in gd:
      d.wait()
    sd = [pltpu.async_copy(rows3.at[j], acc.at[didx.at[j]], ssem, add=True)
          for j in range(MACRO)]
    for d in sd:
      d.wait()
  plsc.subcore_barrier()

  bvec = gpb[0, :]
  gwv = gpb[1, :]
  gbv = gpb[2, :]
  gmv = gpb[3, :]

  nbase = s * RT

  def stats_chunk(rb, cs, bic):
    pltpu.sync_copy(acc.at[pl.ds(rb, 128)], vrow)
    pltpu.sync_copy(dinv.at[pl.ds(rb, 128)], dbc)
    pltpu.sync_copy(batch.at[pl.ds(rb, 128)], bic128)
    if cs != 128:
      for q in range(cs // 16):
        bic[pl.ds(q * 16, 16)] = bic128[pl.ds(q * 16, 16)]

    @pl.loop(0, cs // 16)
    def _(i16):
      b0 = i16 * 16
      dvec = dbc[pl.ds(b0, 16)]
      for k in range(16):
        h = vrow[b0 + k, :] * dvec[k] + bvec
        hh2[b0 + k, pl.ds(0, 16)] = h
        hh2[b0 + k, pl.ds(16, 16)] = h * h

    pltpu.sync_copy(hh2.at[pl.ds(0, cs)], mom.at[bic], add=True)

  nfull = jnp.where(s == NSUB - 1, 46, 49)

  @pl.loop(0, nfull)
  def _(j):
    stats_chunk(nbase + j * 128, 128, bic128)

  @pl.when(s == NSUB - 1)
  def _():
    stats_chunk(nbase + 46 * 128, 32, bic32)
  plsc.subcore_barrier()

  pltpu.sync_copy(mom.at[pl.ds(s * 32, 32)], mrows)
  pltpu.sync_copy(invc, icb)

  @pl.loop(0, 2)
  def _(g16):
    g0 = g16 * 16
    icv = icb[pl.ds(s * 32 + g0, 16)]
    for k in range(16):
      ic = icv[k]
      mean = mrows[g0 + k, pl.ds(0, 16)] * ic
      m2 = mrows[g0 + k, pl.ds(16, 16)] * ic
      var = m2 - (2.0 * gmv - gmv * gmv) * mean * mean
      prmb[g0 + k, pl.ds(0, 16)] = gmv * mean
      prmb[g0 + k, pl.ds(16, 16)] = gwv * _rsqrt16(var + EPS)

  pltpu.sync_copy(prmb, prm.at[pl.ds(s * 32, 32)])
  plsc.subcore_barrier()

  def final_chunk(rb, cs, bic):
    pltpu.sync_copy(acc.at[pl.ds(rb, 128)], vrow)
    pltpu.sync_copy(dinv.at[pl.ds(rb, 128)], dbc)
    pltpu.sync_copy(batch.at[pl.ds(rb, 128)], bic128)
    if cs != 128:
      for q in range(cs // 16):
        bic[pl.ds(q * 16, 16)] = bic128[pl.ds(q * 16, 16)]
    pltpu.sync_copy(prm.at[bic128], pbuf)
    if has_res:
      pltpu.sync_copy(xp.at[c].at[pl.ds(rb, 128)], xpb)

    @pl.loop(0, cs // 16)
    def _(i16):
      b0 = i16 * 16
      dvec = dbc[pl.ds(b0, 16)]
      for k in range(16):
        h = vrow[b0 + k, :] * dvec[k] + bvec
        o = ((h - pbuf[b0 + k, pl.ds(0, 16)]) * pbuf[b0 + k, pl.ds(16, 16)]
             + gbv)
        if has_res:
          o = o + xpb[b0 + k, :]
        xb[b0 + k, :] = jnp.maximum(o, 0.0)

    pltpu.sync_copy(xb, xo.at[c].at[pl.ds(rb, 128)])
    if do_pool:
      pltpu.sync_copy(xb.at[pl.ds(0, cs)], psh.at[bic], add=True)

  @pl.loop(0, nfull)
  def _(j):
    final_chunk(nbase + j * 128, 128, bic128)

  @pl.when(s == NSUB - 1)
  def _():
    final_chunk(nbase + 46 * 128, 32, bic32)

  if do_pool:
    plsc.subcore_barrier()
    pltpu.sync_copy(psh.at[pl.ds(s * 32, 32)], poolo.at[c, pl.ds(s * 32, 32)])


def _layer(y, srcp, dstp, dinv, invc, batch, gp, xp, do_pool):
  has_res = xp is not None
  out_type = [jax.ShapeDtypeStruct((2, NPAD, HH), jnp.float32)]
  if do_pool:
    out_type.append(jax.ShapeDtypeStruct((2, G, HH), jnp.float32))
  ker = pl.kernel(
      functools.partial(_layer_body, has_res, do_pool),
      out_type=out_type,
      mesh=_mesh(),
      compiler_params=_sc_params(),
      scratch_types=[
          pltpu.VMEM_SHARED((NPAD, HH), jnp.float32),
          pltpu.VMEM_SHARED((G, 2 * HH), jnp.float32),
          pltpu.VMEM_SHARED((G, 2 * HH), jnp.float32),
          pltpu.VMEM_SHARED((G, HH), jnp.float32),
          pltpu.VMEM((MACRO, 128), jnp.int32),
          pltpu.VMEM((MACRO, 128), jnp.int32),
          pltpu.VMEM((MACRO, 128, HH), jnp.float32),
          pltpu.VMEM((128, HH), jnp.float32),
          pltpu.VMEM((128, 2 * HH), jnp.float32),
          pltpu.VMEM((128,), jnp.float32),
          pltpu.VMEM((128,), jnp.int32),
          pltpu.VMEM((32,), jnp.int32),
          pltpu.VMEM((128, 2 * HH), jnp.float32),
          pltpu.VMEM((128, HH), jnp.float32),
          pltpu.VMEM((128, HH), jnp.float32),
          pltpu.VMEM((32, 2 * HH), jnp.float32),
          pltpu.VMEM((G,), jnp.float32),
          pltpu.VMEM((32, 2 * HH), jnp.float32),
          pltpu.VMEM((4, HH), jnp.float32),
          pltpu.SemaphoreType.DMA,
          pltpu.SemaphoreType.DMA,
      ],
  )
  args = [y, srcp, dstp, dinv, invc, batch, gp]
  if has_res:
    args.append(xp)
  return ker(*args)


def kernel(x, edge_index, batch, W1, b1, g1w, g1b, g1m, W2, b2, g2w, g2b, g2m,
           W3, b3, g3w, g3b, g3m, lW, lb):
  src = edge_index[0]
  dst = edge_index[1]
  srcp = jnp.concatenate(
      [src, jnp.zeros((EPAD,), jnp.int32)]).reshape(EROWS, 128)
  dstp = jnp.concatenate(
      [dst, jnp.full((EPAD,), DUMP, jnp.int32)]).reshape(EROWS, 128)

  def gpk(b, gw, gb, gm):
    stacked = jnp.stack([b.reshape(2, HH), gw.reshape(2, HH),
                         gb.reshape(2, HH), gm.reshape(2, HH)])
    return stacked.transpose(1, 0, 2)

  degp = _deg_kernel(dstp)[:, 0, :N]
  y1, dinv, invc = _b1(x, degp, batch, W1)
  dinv_p = jnp.concatenate([dinv, jnp.zeros((NPAD - N,), jnp.float32)])
  batch_p = jnp.concatenate([batch, jnp.zeros((NPAD - N,), jnp.int32)])
  (x1,) = _layer(y1, srcp, dstp, dinv_p, invc, batch_p,
                 gpk(b1, g1w, g1b, g1m), None, False)
  y2 = _b(x1, dinv, W2)
  (x2,) = _layer(y2, srcp, dstp, dinv_p, invc, batch_p,
                 gpk(b2, g2w, g2b, g2m), x1, False)
  y3 = _b(x2, dinv, W3)
  x3, poolp = _layer(y3, srcp, dstp, dinv_p, invc, batch_p,
                     gpk(b3, g3w, g3b, g3m), x2, True)
  del x3
  return _f(poolp, invc, lW, lb)


# speedup vs baseline: 24.2218x; 1.1276x over previous
import dataclasses
import functools

import jax
import jax.numpy as jnp
from jax import lax
from jax.experimental import pallas as pl
from jax.experimental.pallas import tpu as pltpu
from jax.experimental.pallas import tpu_sc as plsc

N = 100000
E = 3200000
G = 512
H = 32
HH = 16
EPS = 1e-5

NSUB = 16
EROWS = 25088
EPAD = EROWS * 128 - E
NPAD = 100096
DUMP = N
RT = 6272
RT15 = 5920
MACRO = 16

@functools.cache
def _mesh():
  return plsc.VectorSubcoreMesh(core_axis_name="c", subcore_axis_name="s")


def _sc_params():
  cp = pltpu.CompilerParams()
  cp = dataclasses.replace(cp, needs_layout_passes=False,
                           use_tc_tiling_on_sc=False)
  return cp


def _zero_vmem(ref, rows, cols):
  @pl.loop(0, rows)
  def _(i):
    @pl.loop(0, cols, step=16)
    def _(j):
      ref[i, pl.ds(j, 16)] = jnp.zeros((16,), jnp.float32)


def _rsqrt16(x):
  i = plsc.bitcast(x, jnp.int32)
  i = jnp.int32(0x5F3759DF) - lax.shift_right_logical(i, 1)
  r = plsc.bitcast(i, jnp.float32)
  for _ in range(3):
    r = r * (1.5 - 0.5 * x * r * r)
  return r


def _deg_body(dstp, degp, acc, idxb, ones, zb, sems):
  c = lax.axis_index("c")
  s = lax.axis_index("s")
  @pl.loop(0, RT, step=16)
  def _(j):
    zb[pl.ds(j, 16)] = jnp.zeros((16,), jnp.float32)

  @pl.when(s < NSUB - 1)
  def _():
    pltpu.sync_copy(zb, acc.at[pl.ds(s * RT, RT)])

  @pl.when(s == NSUB - 1)
  def _():
    pltpu.sync_copy(zb.at[pl.ds(0, NPAD - (NSUB - 1) * RT)],
                    acc.at[pl.ds((NSUB - 1) * RT, NPAD - (NSUB - 1) * RT)])
  plsc.subcore_barrier()
  wid = c * NSUB + s
  base = wid * (EROWS // 32)
  @pl.loop(0, (EROWS // 32) // MACRO)
  def _(m):
    pltpu.sync_copy(dstp.at[pl.ds(base + m * MACRO, MACRO)], idxb)
    descs = []
    for j in range(MACRO):
      descs.append(
          pltpu.async_copy(ones, acc.at[idxb.at[j]], sems, add=True))
    for d in descs:
      d.wait()
  plsc.subcore_barrier()

  @pl.when(s < NSUB - 1)
  def _():
    pltpu.sync_copy(acc.at[pl.ds(s * RT, RT)],
                    degp.at[c, 0, pl.ds(s * RT, RT)])

  @pl.when(s == NSUB - 1)
  def _():
    pltpu.sync_copy(acc.at[pl.ds((NSUB - 1) * RT, NPAD - (NSUB - 1) * RT)],
                    degp.at[c, 0, pl.ds((NSUB - 1) * RT,
                                        NPAD - (NSUB - 1) * RT)])


def _deg_kernel(dstp):
  ker = pl.kernel(
      _deg_body,
      out_type=jax.ShapeDtypeStruct((2, 1, NPAD), jnp.float32),
      mesh=_mesh(),
      compiler_params=_sc_params(),
      scratch_types=[
          pltpu.VMEM_SHARED((NPAD,), jnp.float32),
          pltpu.VMEM((MACRO, 128), jnp.int32),
          pltpu.VMEM((128,), jnp.float32),
          pltpu.VMEM((RT,), jnp.float32),
          pltpu.SemaphoreType.DMA,
      ],
  )
  return ker(dstp)


def _b1_body(x_ref, degp_ref, batch_ref, w_ref, y_ref, dinv_ref, invc_ref,
             cnt_ref):
  i = pl.program_id(0)

  @pl.when(i == 0)
  def _():
    cnt_ref[...] = jnp.zeros_like(cnt_ref)

  deg = degp_ref[0, 0, :] + degp_ref[0, 1, :] + 1.0
  dinv = lax.rsqrt(deg)
  dinv_ref[0, 0, :] = dinv
  xw = jnp.dot(x_ref[...], w_ref[...], preferred_element_type=jnp.float32)
  y = xw * dinv[:, None]
  y_ref[0, :, :] = y[:, :HH]
  y_ref[1, :, :] = y[:, HH:]
  mask = (batch_ref[0, 0, :][:, None] ==
          lax.broadcasted_iota(jnp.int32, (1, G), 1)).astype(jnp.float32)
  cnt_ref[...] += jnp.sum(mask, axis=0)

  @pl.when(i == pl.num_programs(0) - 1)
  def _():
    invc_ref[...] = 1.0 / cnt_ref[...]


def _b1(x, degp, batch, w1):
  R = 5000
  grid = N // R
  degp_t = degp.reshape(2, grid, R).transpose(1, 0, 2)
  batch2 = batch.reshape(grid, 1, R)
  y, dinv2, invc = pl.pallas_call(
      _b1_body,
      grid=(grid,),
      in_specs=[
          pl.BlockSpec((R, 3), lambda i: (i, 0)),
          pl.BlockSpec((1, 2, R), lambda i: (i, 0, 0)),
          pl.BlockSpec((1, 1, R), lambda i: (i, 0, 0)),
          pl.BlockSpec((3, H), lambda i: (0, 0)),
      ],
      out_specs=[
          pl.BlockSpec((2, R, HH), lambda i: (0, i, 0)),
          pl.BlockSpec((1, 1, R), lambda i: (i, 0, 0)),
          pl.BlockSpec((G,), lambda i: (0,)),
      ],
      out_shape=[
          jax.ShapeDtypeStruct((2, NPAD, HH), jnp.float32),
          jax.ShapeDtypeStruct((grid, 1, R), jnp.float32),
          jax.ShapeDtypeStruct((G,), jnp.float32),
      ],
      scratch_shapes=[pltpu.VMEM((G,), jnp.float32)],
  )(x, degp_t, batch2, w1)
  return y, dinv2.reshape(N), invc


def _b_body(x_ref, dinv_ref, w_ref, y_ref):
  xc = jnp.concatenate([x_ref[0, :, :], x_ref[1, :, :]], axis=1)
  xw = jnp.dot(xc, w_ref[...], preferred_element_type=jnp.float32)
  y = xw * dinv_ref[0, 0, :][:, None]
  y_ref[0, :, :] = y[:, :HH]
  y_ref[1, :, :] = y[:, HH:]


def _b(x2, dinv, w):
  R = 5000
  grid = N // R
  return pl.pallas_call(
      _b_body,
      grid=(grid,),
      in_specs=[
          pl.BlockSpec((2, R, HH), lambda i: (0, i, 0)),
          pl.BlockSpec((1, 1, R), lambda i: (i, 0, 0)),
          pl.BlockSpec((H, H), lambda i: (0, 0)),
      ],
      out_specs=pl.BlockSpec((2, R, HH), lambda i: (0, i, 0)),
      out_shape=jax.ShapeDtypeStruct((2, NPAD, HH), jnp.float32),
  )(x2, dinv.reshape(grid, 1, R), w)


def _f_body(poolp_ref, invc_ref, lw_ref, lb_ref, o_ref):
  pooled = jnp.concatenate([poolp_ref[0, :, :], poolp_ref[1, :, :]], axis=1)
  pooled = pooled * invc_ref[...][:, None]
  o_ref[...] = (jnp.dot(pooled, lw_ref[...],
                        preferred_element_type=jnp.float32) +
                lb_ref[...][None, :])


def _f(poolp, invc, lw, lb):
  return pl.pallas_call(
      _f_body,
      out_shape=jax.ShapeDtypeStruct((G, 3), jnp.float32),
  )(poolp, invc, lw, lb)


def _layer_body(has_res, do_pool, *refs):
  if has_res:
    (y, srcp, dstp, dinv, invc, batch, gp, xp, xo, *rest) = refs
  else:
    (y, srcp, dstp, dinv, invc, batch, gp, xo, *rest) = refs
    xp = None
  if do_pool:
    poolo = rest[0]
    rest = rest[1:]
  else:
    poolo = None
  (acc, mom, prm, psh, sidx, didx, rba, rbb, vrow, hh2, dbc, bic128, bic32,
   xpb, xb, mrows, icb, prmb, gpb, gsem, ssem) = rest
  pbuf = hh2

  c = lax.axis_index("c")
  s = lax.axis_index("s")

  @pl.when(s < NSUB - 1)
  def _():
    pltpu.sync_copy(y.at[c].at[pl.ds(s * RT, RT)], acc.at[pl.ds(s * RT, RT)])

  @pl.when(s == NSUB - 1)
  def _():
    pltpu.sync_copy(y.at[c].at[pl.ds((NSUB - 1) * RT, NPAD - (NSUB - 1) * RT)],
                    acc.at[pl.ds((NSUB - 1) * RT, NPAD - (NSUB - 1) * RT)])

  _zero_vmem(hh2, 32, 32)
  pltpu.sync_copy(hh2.at[pl.ds(0, 32)], mom.at[pl.ds(s * 32, 32)])
  if do_pool:
    _zero_vmem(xb, 32, 16)
    pltpu.sync_copy(xb.at[pl.ds(0, 32)], psh.at[pl.ds(s * 32, 32)])
  pltpu.sync_copy(gp.at[c], gpb)
  plsc.subcore_barrier()

  ebase = s * (EROWS // NSUB)
  @pl.loop(0, (EROWS // NSUB) // MACRO)
  def _(m):
    pltpu.sync_copy(srcp.at[pl.ds(ebase + m * MACRO, MACRO)], sidx)
    pltpu.sync_copy(dstp.at[pl.ds(ebase + m * MACRO, MACRO)], didx)
    sdesc = {}
    for w in range(MACRO // 2):
      buf = rba if w % 2 == 0 else rbb
      if w >= 2:
        for d in sdesc[w - 2]:
          d.wait()
      gd = [pltpu.async_copy(y.at[c].at[sidx.at[2 * w + j]], buf.at[j], gsem)
            for j in range(2)]
      for d in gd:
        d.wait()
      sdesc[w] = [
          pltpu.async_copy(buf.at[j], acc.at[didx.at[2 * w + j]], ssem,
                           add=True) for j in range(2)]
    for w in (MACRO // 2 - 2, MACRO // 2 - 1):
      for d in sdesc[w]:
        d.wait()
  plsc.subcore_barrier()

  bvec = gpb[0, :]
  gwv = gpb[1, :]
  gbv = gpb[2, :]
  gmv = gpb[3, :]

  nbase = s * RT

  def stats_chunk(rb, cs, bic):
    pltpu.sync_copy(acc.at[pl.ds(rb, 128)], vrow)
    pltpu.sync_copy(dinv.at[pl.ds(rb, 128)], dbc)
    pltpu.sync_copy(batch.at[pl.ds(rb, 128)], bic128)
    if cs != 128:
      for q in range(cs // 16):
        bic[pl.ds(q * 16, 16)] = bic128[pl.ds(q * 16, 16)]

    @pl.loop(0, cs // 16)
    def _(i16):
      b0 = i16 * 16
      dvec = dbc[pl.ds(b0, 16)]
      for k in range(16):
        h = vrow[b0 + k, :] * dvec[k] + bvec
        hh2[b0 + k, pl.ds(0, 16)] = h
        hh2[b0 + k, pl.ds(16, 16)] = h * h

    pltpu.sync_copy(hh2.at[pl.ds(0, cs)], mom.at[bic], add=True)

  nfull = jnp.where(s == NSUB - 1, 46, 49)

  @pl.loop(0, nfull)
  def _(j):
    stats_chunk(nbase + j * 128, 128, bic128)

  @pl.when(s == NSUB - 1)
  def _():
    stats_chunk(nbase + 46 * 128, 32, bic32)
  plsc.subcore_barrier()

  pltpu.sync_copy(mom.at[pl.ds(s * 32, 32)], mrows)
  pltpu.sync_copy(invc, icb)

  @pl.loop(0, 2)
  def _(g16):
    g0 = g16 * 16
    icv = icb[pl.ds(s * 32 + g0, 16)]
    for k in range(16):
      ic = icv[k]
      mean = mrows[g0 + k, pl.ds(0, 16)] * ic
      m2 = mrows[g0 + k, pl.ds(16, 16)] * ic
      var = m2 - (2.0 * gmv - gmv * gmv) * mean * mean
      prmb[g0 + k, pl.ds(0, 16)] = gmv * mean
      prmb[g0 + k, pl.ds(16, 16)] = gwv * _rsqrt16(var + EPS)

  pltpu.sync_copy(prmb, prm.at[pl.ds(s * 32, 32)])
  plsc.subcore_barrier()

  def final_chunk(rb, cs, bic):
    pltpu.sync_copy(acc.at[pl.ds(rb, 128)], vrow)
    pltpu.sync_copy(dinv.at[pl.ds(rb, 128)], dbc)
    pltpu.sync_copy(batch.at[pl.ds(rb, 128)], bic128)
    if cs != 128:
      for q in range(cs // 16):
        bic[pl.ds(q * 16, 16)] = bic128[pl.ds(q * 16, 16)]
    pltpu.sync_copy(prm.at[bic128], pbuf)
    if has_res:
      pltpu.sync_copy(xp.at[c].at[pl.ds(rb, 128)], xpb)

    @pl.loop(0, cs // 16)
    def _(i16):
      b0 = i16 * 16
      dvec = dbc[pl.ds(b0, 16)]
      for k in range(16):
        h = vrow[b0 + k, :] * dvec[k] + bvec
        o = ((h - pbuf[b0 + k, pl.ds(0, 16)]) * pbuf[b0 + k, pl.ds(16, 16)]
             + gbv)
        if has_res:
          o = o + xpb[b0 + k, :]
        xb[b0 + k, :] = jnp.maximum(o, 0.0)

    pltpu.sync_copy(xb, xo.at[c].at[pl.ds(rb, 128)])
    if do_pool:
      pltpu.sync_copy(xb.at[pl.ds(0, cs)], psh.at[bic], add=True)

  @pl.loop(0, nfull)
  def _(j):
    final_chunk(nbase + j * 128, 128, bic128)

  @pl.when(s == NSUB - 1)
  def _():
    final_chunk(nbase + 46 * 128, 32, bic32)

  if do_pool:
    plsc.subcore_barrier()
    pltpu.sync_copy(psh.at[pl.ds(s * 32, 32)], poolo.at[c, pl.ds(s * 32, 32)])


def _layer(y, srcp, dstp, dinv, invc, batch, gp, xp, do_pool):
  has_res = xp is not None
  out_type = [jax.ShapeDtypeStruct((2, NPAD, HH), jnp.float32)]
  if do_pool:
    out_type.append(jax.ShapeDtypeStruct((2, G, HH), jnp.float32))
  ker = pl.kernel(
      functools.partial(_layer_body, has_res, do_pool),
      out_type=out_type,
      mesh=_mesh(),
      compiler_params=_sc_params(),
      scratch_types=[
          pltpu.VMEM_SHARED((NPAD, HH), jnp.float32),
          pltpu.VMEM_SHARED((G, 2 * HH), jnp.float32),
          pltpu.VMEM_SHARED((G, 2 * HH), jnp.float32),
          pltpu.VMEM_SHARED((G, HH), jnp.float32),
          pltpu.VMEM((MACRO, 128), jnp.int32),
          pltpu.VMEM((MACRO, 128), jnp.int32),
          pltpu.VMEM((2, 128, HH), jnp.float32),
          pltpu.VMEM((2, 128, HH), jnp.float32),
          pltpu.VMEM((128, HH), jnp.float32),
          pltpu.VMEM((128, 2 * HH), jnp.float32),
          pltpu.VMEM((128,), jnp.float32),
          pltpu.VMEM((128,), jnp.int32),
          pltpu.VMEM((32,), jnp.int32),
          pltpu.VMEM((128, HH), jnp.float32),
          pltpu.VMEM((128, HH), jnp.float32),
          pltpu.VMEM((32, 2 * HH), jnp.float32),
          pltpu.VMEM((G,), jnp.float32),
          pltpu.VMEM((32, 2 * HH), jnp.float32),
          pltpu.VMEM((4, HH), jnp.float32),
          pltpu.SemaphoreType.DMA,
          pltpu.SemaphoreType.DMA,
      ],
  )
  args = [y, srcp, dstp, dinv, invc, batch, gp]
  if has_res:
    args.append(xp)
  return ker(*args)


def kernel(x, edge_index, batch, W1, b1, g1w, g1b, g1m, W2, b2, g2w, g2b, g2m,
           W3, b3, g3w, g3b, g3m, lW, lb):
  src = edge_index[0]
  dst = edge_index[1]
  srcp = jnp.concatenate(
      [src, jnp.zeros((EPAD,), jnp.int32)]).reshape(EROWS, 128)
  dstp = jnp.concatenate(
      [dst, jnp.full((EPAD,), DUMP, jnp.int32)]).reshape(EROWS, 128)

  def gpk(b, gw, gb, gm):
    stacked = jnp.stack([b.reshape(2, HH), gw.reshape(2, HH),
                         gb.reshape(2, HH), gm.reshape(2, HH)])
    return stacked.transpose(1, 0, 2)

  degp = _deg_kernel(dstp)[:, 0, :N]
  y1, dinv, invc = _b1(x, degp, batch, W1)
  dinv_p = jnp.concatenate([dinv, jnp.zeros((NPAD - N,), jnp.float32)])
  batch_p = jnp.concatenate([batch, jnp.zeros((NPAD - N,), jnp.int32)])
  (x1,) = _layer(y1, srcp, dstp, dinv_p, invc, batch_p,
                 gpk(b1, g1w, g1b, g1m), None, False)
  y2 = _b(x1, dinv, W2)
  (x2,) = _layer(y2, srcp, dstp, dinv_p, invc, batch_p,
                 gpk(b2, g2w, g2b, g2m), x1, False)
  y3 = _b(x2, dinv, W3)
  x3, poolp = _layer(y3, srcp, dstp, dinv_p, invc, batch_p,
                     gpk(b3, g3w, g3b, g3m), x2, True)
  del x3
  return _f(poolp, invc, lW, lb)


# speedup vs baseline: 24.2268x; 1.0002x over previous
import dataclasses
import functools

import jax
import jax.numpy as jnp
from jax import lax
from jax.experimental import pallas as pl
from jax.experimental.pallas import tpu as pltpu
from jax.experimental.pallas import tpu_sc as plsc

N = 100000
E = 3200000
G = 512
H = 32
HH = 16
EPS = 1e-5

NSUB = 16
EROWS = 25088
EPAD = EROWS * 128 - E
NPAD = 100096
DUMP = N
RT = 6272
RT15 = 5920
MACRO = 16

@functools.cache
def _mesh():
  return plsc.VectorSubcoreMesh(core_axis_name="c", subcore_axis_name="s")


def _sc_params():
  cp = pltpu.CompilerParams()
  cp = dataclasses.replace(cp, needs_layout_passes=False,
                           use_tc_tiling_on_sc=False)
  return cp


def _zero_vmem(ref, rows, cols):
  @pl.loop(0, rows)
  def _(i):
    @pl.loop(0, cols, step=16)
    def _(j):
      ref[i, pl.ds(j, 16)] = jnp.zeros((16,), jnp.float32)


def _rsqrt16(x):
  i = plsc.bitcast(x, jnp.int32)
  i = jnp.int32(0x5F3759DF) - lax.shift_right_logical(i, 1)
  r = plsc.bitcast(i, jnp.float32)
  for _ in range(3):
    r = r * (1.5 - 0.5 * x * r * r)
  return r


def _deg_body(dstp, degp, acc, idxb, ones, zb, sems):
  c = lax.axis_index("c")
  s = lax.axis_index("s")
  @pl.loop(0, RT, step=16)
  def _(j):
    zb[pl.ds(j, 16)] = jnp.zeros((16,), jnp.float32)

  @pl.when(s < NSUB - 1)
  def _():
    pltpu.sync_copy(zb, acc.at[pl.ds(s * RT, RT)])

  @pl.when(s == NSUB - 1)
  def _():
    pltpu.sync_copy(zb.at[pl.ds(0, NPAD - (NSUB - 1) * RT)],
                    acc.at[pl.ds((NSUB - 1) * RT, NPAD - (NSUB - 1) * RT)])
  plsc.subcore_barrier()
  wid = c * NSUB + s
  base = wid * (EROWS // 32)
  @pl.loop(0, (EROWS // 32) // MACRO)
  def _(m):
    pltpu.sync_copy(dstp.at[pl.ds(base + m * MACRO, MACRO)], idxb)
    descs = []
    for j in range(MACRO):
      descs.append(
          pltpu.async_copy(ones, acc.at[idxb.at[j]], sems, add=True))
    for d in descs:
      d.wait()
  plsc.subcore_barrier()

  @pl.when(s < NSUB - 1)
  def _():
    pltpu.sync_copy(acc.at[pl.ds(s * RT, RT)],
                    degp.at[c, 0, pl.ds(s * RT, RT)])

  @pl.when(s == NSUB - 1)
  def _():
    pltpu.sync_copy(acc.at[pl.ds((NSUB - 1) * RT, NPAD - (NSUB - 1) * RT)],
                    degp.at[c, 0, pl.ds((NSUB - 1) * RT,
                                        NPAD - (NSUB - 1) * RT)])


def _deg_kernel(dstp):
  ker = pl.kernel(
      _deg_body,
      out_type=jax.ShapeDtypeStruct((2, 1, NPAD), jnp.float32),
      mesh=_mesh(),
      compiler_params=_sc_params(),
      scratch_types=[
          pltpu.VMEM_SHARED((NPAD,), jnp.float32),
          pltpu.VMEM((MACRO, 128), jnp.int32),
          pltpu.VMEM((128,), jnp.float32),
          pltpu.VMEM((RT,), jnp.float32),
          pltpu.SemaphoreType.DMA,
      ],
  )
  return ker(dstp)


def _b1_body(x_ref, degp_ref, batch_ref, w_ref, y_ref, dinv_ref, invc_ref,
             cnt_ref):
  i = pl.program_id(0)

  @pl.when(i == 0)
  def _():
    cnt_ref[...] = jnp.zeros_like(cnt_ref)

  deg = degp_ref[0, 0, :] + degp_ref[0, 1, :] + 1.0
  dinv = lax.rsqrt(deg)
  dinv_ref[0, 0, :] = dinv
  xw = jnp.dot(x_ref[...], w_ref[...], preferred_element_type=jnp.float32)
  y = xw * dinv[:, None]
  y_ref[0, :, :] = y[:, :HH]
  y_ref[1, :, :] = y[:, HH:]
  mask = (batch_ref[0, 0, :][:, None] ==
          lax.broadcasted_iota(jnp.int32, (1, G), 1)).astype(jnp.float32)
  cnt_ref[...] += jnp.sum(mask, axis=0)

  @pl.when(i == pl.num_programs(0) - 1)
  def _():
    invc_ref[...] = 1.0 / cnt_ref[...]


def _b1(x, degp, batch, w1):
  R = 5000
  grid = N // R
  degp_t = degp.reshape(2, grid, R).transpose(1, 0, 2)
  batch2 = batch.reshape(grid, 1, R)
  y, dinv2, invc = pl.pallas_call(
      _b1_body,
      grid=(grid,),
      in_specs=[
          pl.BlockSpec((R, 3), lambda i: (i, 0)),
          pl.BlockSpec((1, 2, R), lambda i: (i, 0, 0)),
          pl.BlockSpec((1, 1, R), lambda i: (i, 0, 0)),
          pl.BlockSpec((3, H), lambda i: (0, 0)),
      ],
      out_specs=[
          pl.BlockSpec((2, R, HH), lambda i: (0, i, 0)),
          pl.BlockSpec((1, 1, R), lambda i: (i, 0, 0)),
          pl.BlockSpec((G,), lambda i: (0,)),
      ],
      out_shape=[
          jax.ShapeDtypeStruct((2, NPAD, HH), jnp.float32),
          jax.ShapeDtypeStruct((grid, 1, R), jnp.float32),
          jax.ShapeDtypeStruct((G,), jnp.float32),
      ],
      scratch_shapes=[pltpu.VMEM((G,), jnp.float32)],
  )(x, degp_t, batch2, w1)
  return y, dinv2.reshape(N), invc


def _b_body(x_ref, dinv_ref, w_ref, y_ref):
  xc = jnp.concatenate([x_ref[0, :, :], x_ref[1, :, :]], axis=1)
  xw = jnp.dot(xc, w_ref[...], preferred_element_type=jnp.float32)
  y = xw * dinv_ref[0, 0, :][:, None]
  y_ref[0, :, :] = y[:, :HH]
  y_ref[1, :, :] = y[:, HH:]


def _b(x2, dinv, w):
  R = 5000
  grid = N // R
  return pl.pallas_call(
      _b_body,
      grid=(grid,),
      in_specs=[
          pl.BlockSpec((2, R, HH), lambda i: (0, i, 0)),
          pl.BlockSpec((1, 1, R), lambda i: (i, 0, 0)),
          pl.BlockSpec((H, H), lambda i: (0, 0)),
      ],
      out_specs=pl.BlockSpec((2, R, HH), lambda i: (0, i, 0)),
      out_shape=jax.ShapeDtypeStruct((2, NPAD, HH), jnp.float32),
  )(x2, dinv.reshape(grid, 1, R), w)


def _f_body(poolp_ref, invc_ref, lw_ref, lb_ref, o_ref):
  pooled = jnp.concatenate([poolp_ref[0, :, :], poolp_ref[1, :, :]], axis=1)
  pooled = pooled * invc_ref[...][:, None]
  o_ref[...] = (jnp.dot(pooled, lw_ref[...],
                        preferred_element_type=jnp.float32) +
                lb_ref[...][None, :])


def _f(poolp, invc, lw, lb):
  return pl.pallas_call(
      _f_body,
      out_shape=jax.ShapeDtypeStruct((G, 3), jnp.float32),
  )(poolp, invc, lw, lb)


def _layer_body(has_res, do_pool, *refs):
  if has_res:
    (y, srcp, dstp, dinv, invc, batch, gp, xp, xo, *rest) = refs
  else:
    (y, srcp, dstp, dinv, invc, batch, gp, xo, *rest) = refs
    xp = None
  if do_pool:
    poolo = rest[0]
    rest = rest[1:]
  else:
    poolo = None
  (acc, mom, prm, psh, sidx, didx, rba, rbb, vrow, hh2, dbc, bic128, bic32,
   xpb, xb, mrows, icb, prmb, gpb, gsem, ssem, ssem2) = rest
  pbuf = hh2

  c = lax.axis_index("c")
  s = lax.axis_index("s")

  @pl.when(s < NSUB - 1)
  def _():
    pltpu.sync_copy(y.at[c].at[pl.ds(s * RT, RT)], acc.at[pl.ds(s * RT, RT)])

  @pl.when(s == NSUB - 1)
  def _():
    pltpu.sync_copy(y.at[c].at[pl.ds((NSUB - 1) * RT, NPAD - (NSUB - 1) * RT)],
                    acc.at[pl.ds((NSUB - 1) * RT, NPAD - (NSUB - 1) * RT)])

  _zero_vmem(hh2, 32, 32)
  pltpu.sync_copy(hh2.at[pl.ds(0, 32)], mom.at[pl.ds(s * 32, 32)])
  if do_pool:
    _zero_vmem(xb, 32, 16)
    pltpu.sync_copy(xb.at[pl.ds(0, 32)], psh.at[pl.ds(s * 32, 32)])
  pltpu.sync_copy(gp.at[c], gpb)
  plsc.subcore_barrier()

  ebase = s * (EROWS // NSUB)
  @pl.loop(0, (EROWS // NSUB) // MACRO)
  def _(m):
    pltpu.sync_copy(srcp.at[pl.ds(ebase + m * MACRO, MACRO)], sidx)
    pltpu.sync_copy(dstp.at[pl.ds(ebase + m * MACRO, MACRO)], didx)
    sdesc = {}
    for w in range(MACRO // 2):
      buf = rba if w % 2 == 0 else rbb
      if w >= 2:
        for d in sdesc[w - 2]:
          d.wait()
      gd = [pltpu.async_copy(y.at[c].at[sidx.at[2 * w + j]], buf.at[j], gsem)
            for j in range(2)]
      for d in gd:
        d.wait()
      wsem = ssem if w % 2 == 0 else ssem2
      sdesc[w] = [
          pltpu.async_copy(buf.at[j], acc.at[didx.at[2 * w + j]], wsem,
                           add=True) for j in range(2)]
    for w in (MACRO // 2 - 2, MACRO // 2 - 1):
      for d in sdesc[w]:
        d.wait()
  plsc.subcore_barrier()

  bvec = gpb[0, :]
  gwv = gpb[1, :]
  gbv = gpb[2, :]
  gmv = gpb[3, :]

  nbase = s * RT

  def stats_chunk(rb, cs, bic):
    pltpu.sync_copy(acc.at[pl.ds(rb, 128)], vrow)
    pltpu.sync_copy(dinv.at[pl.ds(rb, 128)], dbc)
    pltpu.sync_copy(batch.at[pl.ds(rb, 128)], bic128)
    if cs != 128:
      for q in range(cs // 16):
        bic[pl.ds(q * 16, 16)] = bic128[pl.ds(q * 16, 16)]

    @pl.loop(0, cs // 16)
    def _(i16):
      b0 = i16 * 16
      dvec = dbc[pl.ds(b0, 16)]
      for k in range(16):
        h = vrow[b0 + k, :] * dvec[k] + bvec
        hh2[b0 + k, pl.ds(0, 16)] = h
        hh2[b0 + k, pl.ds(16, 16)] = h * h

    pltpu.sync_copy(hh2.at[pl.ds(0, cs)], mom.at[bic], add=True)

  nfull = jnp.where(s == NSUB - 1, 46, 49)

  @pl.loop(0, nfull)
  def _(j):
    stats_chunk(nbase + j * 128, 128, bic128)

  @pl.when(s == NSUB - 1)
  def _():
    stats_chunk(nbase + 46 * 128, 32, bic32)
  plsc.subcore_barrier()

  pltpu.sync_copy(mom.at[pl.ds(s * 32, 32)], mrows)
  pltpu.sync_copy(invc, icb)

  @pl.loop(0, 2)
  def _(g16):
    g0 = g16 * 16
    icv = icb[pl.ds(s * 32 + g0, 16)]
    for k in range(16):
      ic = icv[k]
      mean = mrows[g0 + k, pl.ds(0, 16)] * ic
      m2 = mrows[g0 + k, pl.ds(16, 16)] * ic
      var = m2 - (2.0 * gmv - gmv * gmv) * mean * mean
      prmb[g0 + k, pl.ds(0, 16)] = gmv * mean
      prmb[g0 + k, pl.ds(16, 16)] = gwv * _rsqrt16(var + EPS)

  pltpu.sync_copy(prmb, prm.at[pl.ds(s * 32, 32)])
  plsc.subcore_barrier()

  def final_chunk(rb, cs, bic):
    pltpu.sync_copy(acc.at[pl.ds(rb, 128)], vrow)
    pltpu.sync_copy(dinv.at[pl.ds(rb, 128)], dbc)
    pltpu.sync_copy(batch.at[pl.ds(rb, 128)], bic128)
    if cs != 128:
      for q in range(cs // 16):
        bic[pl.ds(q * 16, 16)] = bic128[pl.ds(q * 16, 16)]
    pltpu.sync_copy(prm.at[bic128], pbuf)
    if has_res:
      pltpu.sync_copy(xp.at[c].at[pl.ds(rb, 128)], xpb)

    @pl.loop(0, cs // 16)
    def _(i16):
      b0 = i16 * 16
      dvec = dbc[pl.ds(b0, 16)]
      for k in range(16):
        h = vrow[b0 + k, :] * dvec[k] + bvec
        o = ((h - pbuf[b0 + k, pl.ds(0, 16)]) * pbuf[b0 + k, pl.ds(16, 16)]
             + gbv)
        if has_res:
          o = o + xpb[b0 + k, :]
        xb[b0 + k, :] = jnp.maximum(o, 0.0)

    pltpu.sync_copy(xb, xo.at[c].at[pl.ds(rb, 128)])
    if do_pool:
      pltpu.sync_copy(xb.at[pl.ds(0, cs)], psh.at[bic], add=True)

  @pl.loop(0, nfull)
  def _(j):
    final_chunk(nbase + j * 128, 128, bic128)

  @pl.when(s == NSUB - 1)
  def _():
    final_chunk(nbase + 46 * 128, 32, bic32)

  if do_pool:
    plsc.subcore_barrier()
    pltpu.sync_copy(psh.at[pl.ds(s * 32, 32)], poolo.at[c, pl.ds(s * 32, 32)])


def _layer(y, srcp, dstp, dinv, invc, batch, gp, xp, do_pool):
  has_res = xp is not None
  out_type = [jax.ShapeDtypeStruct((2, NPAD, HH), jnp.float32)]
  if do_pool:
    out_type.append(jax.ShapeDtypeStruct((2, G, HH), jnp.float32))
  ker = pl.kernel(
      functools.partial(_layer_body, has_res, do_pool),
      out_type=out_type,
      mesh=_mesh(),
      compiler_params=_sc_params(),
      scratch_types=[
          pltpu.VMEM_SHARED((NPAD, HH), jnp.float32),
          pltpu.VMEM_SHARED((G, 2 * HH), jnp.float32),
          pltpu.VMEM_SHARED((G, 2 * HH), jnp.float32),
          pltpu.VMEM_SHARED((G, HH), jnp.float32),
          pltpu.VMEM((MACRO, 128), jnp.int32),
          pltpu.VMEM((MACRO, 128), jnp.int32),
          pltpu.VMEM((2, 128, HH), jnp.float32),
          pltpu.VMEM((2, 128, HH), jnp.float32),
          pltpu.VMEM((128, HH), jnp.float32),
          pltpu.VMEM((128, 2 * HH), jnp.float32),
          pltpu.VMEM((128,), jnp.float32),
          pltpu.VMEM((128,), jnp.int32),
          pltpu.VMEM((32,), jnp.int32),
          pltpu.VMEM((128, HH), jnp.float32),
          pltpu.VMEM((128, HH), jnp.float32),
          pltpu.VMEM((32, 2 * HH), jnp.float32),
          pltpu.VMEM((G,), jnp.float32),
          pltpu.VMEM((32, 2 * HH), jnp.float32),
          pltpu.VMEM((4, HH), jnp.float32),
          pltpu.SemaphoreType.DMA,
          pltpu.SemaphoreType.DMA,
          pltpu.SemaphoreType.DMA,
      ],
  )
  args = [y, srcp, dstp, dinv, invc, batch, gp]
  if has_res:
    args.append(xp)
  return ker(*args)


def kernel(x, edge_index, batch, W1, b1, g1w, g1b, g1m, W2, b2, g2w, g2b, g2m,
           W3, b3, g3w, g3b, g3m, lW, lb):
  src = edge_index[0]
  dst = edge_index[1]
  srcp = jnp.concatenate(
      [src, jnp.zeros((EPAD,), jnp.int32)]).reshape(EROWS, 128)
  dstp = jnp.concatenate(
      [dst, jnp.full((EPAD,), DUMP, jnp.int32)]).reshape(EROWS, 128)

  def gpk(b, gw, gb, gm):
    stacked = jnp.stack([b.reshape(2, HH), gw.reshape(2, HH),
                         gb.reshape(2, HH), gm.reshape(2, HH)])
    return stacked.transpose(1, 0, 2)

  degp = _deg_kernel(dstp)[:, 0, :N]
  y1, dinv, invc = _b1(x, degp, batch, W1)
  dinv_p = jnp.concatenate([dinv, jnp.zeros((NPAD - N,), jnp.float32)])
  batch_p = jnp.concatenate([batch, jnp.zeros((NPAD - N,), jnp.int32)])
  (x1,) = _layer(y1, srcp, dstp, dinv_p, invc, batch_p,
                 gpk(b1, g1w, g1b, g1m), None, False)
  y2 = _b(x1, dinv, W2)
  (x2,) = _layer(y2, srcp, dstp, dinv_p, invc, batch_p,
                 gpk(b2, g2w, g2b, g2m), x1, False)
  y3 = _b(x2, dinv, W3)
  x3, poolp = _layer(y3, srcp, dstp, dinv_p, invc, batch_p,
                     gpk(b3, g3w, g3b, g3m), x2, True)
  del x3
  return _f(poolp, invc, lW, lb)
